# Initial kernel scaffold; baseline (speedup 1.0000x reference)
import jax
import jax.numpy as jnp
from jax.experimental import pallas as pl


def kernel(x, edge_index, batch, params):
    raise NotImplementedError("write your pallas kernel here")


# speedup vs baseline: 4.5083x; 4.5083x over previous
import functools

import jax
import jax.numpy as jnp
from jax import lax
from jax.experimental import pallas as pl
from jax.experimental.pallas import tpu as pltpu
from jax.experimental.pallas import tpu_sc as plsc

_N = 10000
_E = 320000
_G = 64
_NC, _NS = 2, 16
_NW = _NC * _NS
_CHUNK = 128

_EBR = 320
_ESTEPS = 88
_ENP = _NW * _EBR

_PBR = 8
_PSTEPS = 5
_PNP = _NW * _PBR


def _make_sc_segsum(d, steps, br):
    mesh = plsc.VectorSubcoreMesh(core_axis_name="c", subcore_axis_name="s")
    np_rows = _NW * br

    @functools.partial(
        pl.kernel,
        out_type=jax.ShapeDtypeStruct((np_rows, d), jnp.float32),
        mesh=mesh,
        scratch_types=[
            pltpu.VMEM((steps, _CHUNK), jnp.int32),
            pltpu.VMEM((steps, _CHUNK), jnp.int32),
            pltpu.VMEM((_CHUNK, d), jnp.float32),
            pltpu.VMEM_SHARED((np_rows, d), jnp.float32),
            pltpu.SemaphoreType.DMA,
        ],
        compiler_params=pltpu.CompilerParams(use_tc_tiling_on_sc=False),
    )
    def sc_segsum(x_hbm, src_hbm, dst_hbm, zeros_hbm, out_hbm,
                  src_v, dst_v, rows_v, acc_sh, sem):
        cid = lax.axis_index("c")
        sid = lax.axis_index("s")
        wid = sid * _NC + cid
        sl = pl.ds(wid * br, br)
        pltpu.sync_copy(zeros_hbm.at[sl], acc_sh.at[sl])
        pltpu.sync_copy(src_hbm.at[wid], src_v)
        pltpu.sync_copy(dst_hbm.at[wid], dst_v)
        plsc.subcore_barrier()

        def body(j, carry):
            pltpu.async_copy(x_hbm.at[src_v.at[j]], rows_v, sem).wait()
            pltpu.sync_copy(rows_v, acc_sh.at[dst_v.at[j]], add=True)
            return carry

        lax.fori_loop(0, steps, body, 0, unroll=False)
        plsc.subcore_barrier()
        pltpu.sync_copy(acc_sh.at[sl], out_hbm.at[sl])

    return sc_segsum


_make_sc_segsum = functools.lru_cache(maxsize=None)(_make_sc_segsum)


def _make_sc_pool():
    mesh = plsc.VectorSubcoreMesh(core_axis_name="c", subcore_axis_name="s")
    cap = _PSTEPS * _CHUNK

    @functools.partial(
        pl.kernel,
        out_type=jax.ShapeDtypeStruct((_PNP, 64), jnp.float32),
        mesh=mesh,
        scratch_types=[
            pltpu.VMEM((_PSTEPS, _CHUNK), jnp.int32),
            pltpu.VMEM((_PSTEPS, _CHUNK), jnp.int32),
            pltpu.VMEM((cap, 64), jnp.float32),
            pltpu.VMEM((2, 64), jnp.float32),
            pltpu.SemaphoreType.DMA,
        ],
        compiler_params=pltpu.CompilerParams(use_tc_tiling_on_sc=False),
    )
    def sc_pool(x_hbm, src_hbm, dst_hbm, out_hbm,
                src_v, dst_v, rows_v, res_v, sem):
        cid = lax.axis_index("c")
        sid = lax.axis_index("s")
        wid = sid * _NC + cid
        row0 = wid * _PBR
        pltpu.sync_copy(src_hbm.at[wid], src_v)
        pltpu.sync_copy(dst_hbm.at[wid], dst_v)
        for j in range(_PSTEPS):
            pltpu.async_copy(x_hbm.at[src_v.at[j]],
                             rows_v.at[pl.ds(j * _CHUNK, _CHUNK)], sem).wait()

        def body(grp, accs):
            a0, a1 = accs
            a0 = list(a0)
            a1 = list(a1)
            dvec = dst_v[grp // 8, pl.ds((grp % 8) * 16, 16)]
            for t in range(16):
                d = dvec[t]
                is0 = d == row0
                is1 = d == row0 + 1
                r = grp * 16 + t
                for k in range(4):
                    rv = rows_v[r, pl.ds(k * 16, 16)]
                    a0[k] = jnp.where(is0, a0[k] + rv, a0[k])
                    a1[k] = jnp.where(is1, a1[k] + rv, a1[k])
            return tuple(a0), tuple(a1)

        z = jnp.zeros((16,), jnp.float32)
        a0, a1 = lax.fori_loop(0, cap // 16, body,
                               ((z, z, z, z), (z, z, z, z)), unroll=False)
        for k in range(4):
            res_v[0, pl.ds(k * 16, 16)] = a0[k]
            res_v[1, pl.ds(k * 16, 16)] = a1[k]
        pltpu.sync_copy(res_v, out_hbm.at[pl.ds(row0, 2)])

    return sc_pool


_sc_pool_cached = None


def _sc_pool():
    global _sc_pool_cached
    if _sc_pool_cached is None:
        _sc_pool_cached = _make_sc_pool()
    return _sc_pool_cached


def _bucket_edges(src, dst):
    cap = _ESTEPS * _CHUNK
    key = dst // _EBR
    order = jnp.argsort(key, stable=True)
    src_s = src[order]
    dst_s = dst[order]
    key_s = key[order]
    starts = jnp.searchsorted(key_s, jnp.arange(_NW + 1, dtype=jnp.int32))
    pos = starts[:_NW, None] + jnp.arange(cap, dtype=jnp.int32)[None, :]
    valid = pos < starts[1:, None]
    posc = jnp.minimum(pos, _E - 1)
    flat = jnp.arange(_NW * cap, dtype=jnp.int32).reshape(_NW, cap)
    junk_src = flat % _N
    junk_dst = _N + (flat % (_ENP - _N))
    src_t = jnp.where(valid, src_s[posc], junk_src)
    dst_t = jnp.where(valid, dst_s[posc], junk_dst)
    return (src_t.reshape(_NW, _ESTEPS, _CHUNK),
            dst_t.reshape(_NW, _ESTEPS, _CHUNK))


def _bucket_nodes(batch):
    cap = _PSTEPS * _CHUNK
    ss = jnp.searchsorted(batch, jnp.arange(0, _G + 1, 2, dtype=jnp.int32))
    row_of_node = _PBR * (batch // 2) + (batch % 2)
    pos = ss[:_NW, None] + jnp.arange(cap, dtype=jnp.int32)[None, :]
    valid = pos < ss[1:, None]
    posc = jnp.minimum(pos, _N - 1)
    flat = jnp.arange(_NW * cap, dtype=jnp.int32).reshape(_NW, cap)
    junk_src = flat % _N
    b = jnp.arange(_NW, dtype=jnp.int32)[:, None]
    junk_dst = _PBR * b + 2 + (flat % (_PBR - 2))
    src_t = jnp.where(valid, posc, junk_src)
    dst_t = jnp.where(valid, row_of_node[posc], junk_dst)
    return (src_t.reshape(_NW, _PSTEPS, _CHUNK),
            dst_t.reshape(_NW, _PSTEPS, _CHUNK))


def _dot3(h, w):
    hh = h.astype(jnp.bfloat16)
    hl = (h - hh.astype(jnp.float32)).astype(jnp.bfloat16)
    wh = w.astype(jnp.bfloat16)
    wl = (w - wh.astype(jnp.float32)).astype(jnp.bfloat16)

    def mm(a, b):
        return jnp.dot(a, b, preferred_element_type=jnp.float32)

    return (mm(hh, wh) + mm(hh, wl)) + mm(hl, wh)


def _tc_conv_body(x_ref, p_ref, w1, b1, w2, b2, gam, bet, xout_ref):
    h = x_ref[...] + p_ref[: _N, :]
    a = jnp.maximum(jnp.dot(h, w1[...], preferred_element_type=jnp.float32)
                    + b1[...], 0.0)
    a = jnp.maximum(jnp.dot(a, w2[...], preferred_element_type=jnp.float32)
                    + b2[...], 0.0)
    m = jnp.mean(a, axis=0, keepdims=True)
    v = jnp.mean((a - m) * (a - m), axis=0, keepdims=True)
    xout_ref[...] = (a - m) / jnp.sqrt(v + 1e-5) * gam[...] + bet[...]


def _tc_conv(xi, agg, conv_p):
    return pl.pallas_call(
        _tc_conv_body,
        out_shape=jax.ShapeDtypeStruct((_N, 64), jnp.float32),
    )(xi, agg, conv_p['W1'], conv_p['b1'].reshape(1, 64),
      conv_p['W2'], conv_p['b2'].reshape(1, 64),
      conv_p['gamma'].reshape(1, 64), conv_p['beta'].reshape(1, 64))


def _tc_fc_body(pr_ref, fcw, fcb, acc_ref, g_ref, accout_ref):
    pool = pr_ref[...].reshape(_NW, _PBR, 64)[:, :2, :].reshape(_G, 64)
    g = jnp.maximum(
        jnp.dot(acc_ref[...] + pool, fcw[...],
                preferred_element_type=jnp.float32) + fcb[...], 0.0)
    g_ref[...] = g
    accout_ref[...] = acc_ref[...] + g


def _tc_fc(pooled_raw, fc_p, acc):
    out_shape = [
        jax.ShapeDtypeStruct((_G, 64), jnp.float32),
        jax.ShapeDtypeStruct((_G, 64), jnp.float32),
    ]
    return pl.pallas_call(
        _tc_fc_body,
        out_shape=out_shape,
    )(pooled_raw, fc_p['W'], fc_p['b'].reshape(1, 64), acc)


def _tc_head_body(g_ref, lw, lb, out_ref):
    out_ref[...] = (jnp.dot(g_ref[...], lw[...],
                            preferred_element_type=jnp.float32) + lb[...])


def kernel(x, edge_index, batch, params):
    src = edge_index[0].astype(jnp.int32)
    dst = edge_index[1].astype(jnp.int32)
    batch_i = batch.astype(jnp.int32)
    esrc, edst = _bucket_edges(src, dst)
    psrc, pdst = _bucket_nodes(batch_i)
    zeros128 = jnp.zeros((_ENP, 128), jnp.float32)
    zeros64 = jnp.zeros((_ENP, 64), jnp.float32)

    acc = jnp.zeros((_G, 64), jnp.float32)
    xi = x
    g = None
    for i in range(1, 6):
        sc = _make_sc_segsum(128 if i == 1 else 64, _ESTEPS, _EBR)
        zeros = zeros128 if i == 1 else zeros64
        agg = sc(xi, esrc, edst, zeros)
        xi = _tc_conv(xi, agg, params[f'conv{i}'])
        pooled_raw = _sc_pool()(xi, psrc, pdst)
        g, acc = _tc_fc(pooled_raw, params[f'fc{i}'], acc)
    out = pl.pallas_call(
        _tc_head_body,
        out_shape=jax.ShapeDtypeStruct((_G, 1), jnp.float32),
    )(g, params['lin2']['W'], params['lin2']['b'].reshape(1, 1))
    return out.reshape(-1)
